# Initial kernel scaffold; baseline (speedup 1.0000x reference)
import jax
import jax.numpy as jnp
from jax.experimental import pallas as pl


def kernel(X, edge_index, W1, b1, W2, b2):
    raise NotImplementedError("write your pallas kernel here")


# speedup vs baseline: 12.4664x; 12.4664x over previous
import functools

import jax
import jax.numpy as jnp
from jax import lax
from jax.experimental import pallas as pl
from jax.experimental.pallas import tpu as pltpu
from jax.experimental.pallas import tpu_sc as plsc

_NC = 2
_NS = 16
_NW = _NC * _NS
_L = 16
_CHUNK = 128


def _fill_vmem(ref, rows, cols, value):
  vec = jnp.full((_L,), value, jnp.float32)

  def body(i, c):
    for j in range(cols // _L):
      ref[i, pl.ds(j * _L, _L)] = vec
    return c

  lax.fori_loop(0, rows, body, 0)


def _zero_shared_rows(src_v, acc_sh, rbase, rows):
  nfull, rem = divmod(rows, _CHUNK)
  for k in range(nfull):
    pltpu.sync_copy(src_v, acc_sh.at[pl.ds(rbase + k * _CHUNK, _CHUNK)])
  if rem:
    pltpu.sync_copy(
        src_v.at[pl.ds(0, rem)],
        acc_sh.at[pl.ds(rbase + nfull * _CHUNK, rem)],
    )


def _make_deg_kernel(n_pad, chunks_per_tile):
  rows = n_pad // _NS

  @functools.partial(
      pl.kernel,
      out_type=jax.ShapeDtypeStruct((_NC, n_pad, _L), jnp.float32),
      mesh=plsc.VectorSubcoreMesh(core_axis_name="c", subcore_axis_name="s"),
      compiler_params=pltpu.CompilerParams(use_tc_tiling_on_sc=False),
      scratch_types=[
          pltpu.VMEM((_CHUNK,), jnp.int32),
          pltpu.VMEM((_CHUNK, _L), jnp.float32),
          pltpu.VMEM_SHARED((n_pad, _L), jnp.float32),
      ],
  )
  def deg_kernel(dst_hbm, out_hbm, dst_v, val_v, acc_sh):
    cid = lax.axis_index("c")
    sid = lax.axis_index("s")
    wid = sid * _NC + cid
    rbase = sid * rows

    _fill_vmem(val_v, _CHUNK, _L, 0.0)
    _zero_shared_rows(val_v, acc_sh, rbase, rows)
    plsc.subcore_barrier()

    _fill_vmem(val_v, _CHUNK, _L, 1.0)
    ebase = wid * chunks_per_tile

    def body(i, c):
      off = (ebase + i) * _CHUNK
      pltpu.sync_copy(dst_hbm.at[pl.ds(off, _CHUNK)], dst_v)
      pltpu.sync_copy(val_v, acc_sh.at[dst_v], add=True)
      return c

    lax.fori_loop(0, chunks_per_tile, body, 0)
    plsc.subcore_barrier()
    pltpu.sync_copy(
        acc_sh.at[pl.ds(rbase, rows)], out_hbm.at[cid, pl.ds(rbase, rows)]
    )

  return deg_kernel


def _make_agg_kernel(n_pad, d, chunks_per_tile):
  rows = n_pad // _NS

  @functools.partial(
      pl.kernel,
      out_type=jax.ShapeDtypeStruct((_NC, n_pad, d), jnp.float32),
      mesh=plsc.VectorSubcoreMesh(core_axis_name="c", subcore_axis_name="s"),
      compiler_params=pltpu.CompilerParams(use_tc_tiling_on_sc=False),
      scratch_types=[
          pltpu.VMEM((_CHUNK,), jnp.int32),
          pltpu.VMEM((_CHUNK,), jnp.int32),
          pltpu.VMEM((_CHUNK, d), jnp.float32),
          pltpu.VMEM_SHARED((n_pad, d), jnp.float32),
          pltpu.SemaphoreType.DMA,
      ],
  )
  def agg_kernel(y_hbm, src_hbm, dst_hbm, out_hbm, src_v, dst_v, msg_v,
                 acc_sh, sem):
    cid = lax.axis_index("c")
    sid = lax.axis_index("s")
    wid = sid * _NC + cid
    rbase = sid * rows

    _fill_vmem(msg_v, _CHUNK, d, 0.0)
    _zero_shared_rows(msg_v, acc_sh, rbase, rows)
    plsc.subcore_barrier()

    ebase = wid * chunks_per_tile

    def body(i, c):
      off = (ebase + i) * _CHUNK
      pltpu.sync_copy(src_hbm.at[pl.ds(off, _CHUNK)], src_v)
      pltpu.sync_copy(dst_hbm.at[pl.ds(off, _CHUNK)], dst_v)
      pltpu.async_copy(y_hbm.at[src_v], msg_v, sem).wait()
      pltpu.sync_copy(msg_v, acc_sh.at[dst_v], add=True)
      return c

    lax.fori_loop(0, chunks_per_tile, body, 0)
    plsc.subcore_barrier()
    pltpu.sync_copy(
        acc_sh.at[pl.ds(rbase, rows)], out_hbm.at[cid, pl.ds(rbase, rows)]
    )

  return agg_kernel


def _tc_prep(X, W1, degp, blk=1000):
  n, d_in = X.shape
  d_hid = W1.shape[1]

  def body(x_ref, w_ref, d0_ref, d1_ref, y_ref, dinv_ref):
    deg = d0_ref[0][:, 0:1] + d1_ref[0][:, 0:1] + 1.0
    dinv = lax.rsqrt(deg)
    xw = jnp.dot(x_ref[...], w_ref[...], preferred_element_type=jnp.float32)
    y_ref[...] = xw * dinv
    dinv_ref[...] = dinv

  return pl.pallas_call(
      body,
      grid=(n // blk,),
      in_specs=[
          pl.BlockSpec((blk, d_in), lambda i: (i, 0)),
          pl.BlockSpec((d_in, d_hid), lambda i: (0, 0)),
          pl.BlockSpec((1, blk, _L), lambda i: (0, i, 0)),
          pl.BlockSpec((1, blk, _L), lambda i: (1, i, 0)),
      ],
      out_specs=[
          pl.BlockSpec((blk, d_hid), lambda i: (i, 0)),
          pl.BlockSpec((blk, 1), lambda i: (i, 0)),
      ],
      out_shape=[
          jax.ShapeDtypeStruct((n, d_hid), jnp.float32),
          jax.ShapeDtypeStruct((n, 1), jnp.float32),
      ],
  )(X, W1, degp, degp)


def _tc_mid(aggp, y, dinv, b1, W2, blk=1000):
  n, d_hid = y.shape
  d_out = W2.shape[1]

  def body(p0_ref, p1_ref, y_ref, dinv_ref, b1_ref, w2_ref, y2_ref):
    dinv = dinv_ref[...]
    pre = dinv * (p0_ref[0] + p1_ref[0] + y_ref[...]) + b1_ref[...]
    h = jnp.maximum(pre, 0.0)
    z = jnp.dot(h, w2_ref[...], preferred_element_type=jnp.float32)
    y2_ref[...] = z * dinv

  return pl.pallas_call(
      body,
      grid=(n // blk,),
      in_specs=[
          pl.BlockSpec((1, blk, d_hid), lambda i: (0, i, 0)),
          pl.BlockSpec((1, blk, d_hid), lambda i: (1, i, 0)),
          pl.BlockSpec((blk, d_hid), lambda i: (i, 0)),
          pl.BlockSpec((blk, 1), lambda i: (i, 0)),
          pl.BlockSpec((1, d_hid), lambda i: (0, 0)),
          pl.BlockSpec((d_hid, d_out), lambda i: (0, 0)),
      ],
      out_specs=pl.BlockSpec((blk, d_out), lambda i: (i, 0)),
      out_shape=jax.ShapeDtypeStruct((n, d_out), jnp.float32),
  )(aggp, aggp, y, dinv, b1.reshape(1, -1), W2)


def _tc_final(aggp, y2, dinv, b2, blk=1000):
  n, d_out = y2.shape

  def body(q0_ref, q1_ref, y2_ref, dinv_ref, b2_ref, out_ref):
    o = dinv_ref[...] * (q0_ref[0] + q1_ref[0] + y2_ref[...]) + b2_ref[...]
    m = jnp.max(o, axis=1, keepdims=True)
    e = jnp.exp(o - m)
    s = jnp.sum(e, axis=1, keepdims=True)
    out_ref[...] = (o - m) - jnp.log(s)

  return pl.pallas_call(
      body,
      grid=(n // blk,),
      in_specs=[
          pl.BlockSpec((1, blk, d_out), lambda i: (0, i, 0)),
          pl.BlockSpec((1, blk, d_out), lambda i: (1, i, 0)),
          pl.BlockSpec((blk, d_out), lambda i: (i, 0)),
          pl.BlockSpec((blk, 1), lambda i: (i, 0)),
          pl.BlockSpec((1, d_out), lambda i: (0, 0)),
      ],
      out_specs=pl.BlockSpec((blk, d_out), lambda i: (i, 0)),
      out_shape=jax.ShapeDtypeStruct((n, d_out), jnp.float32),
  )(aggp, aggp, y2, dinv, b2.reshape(1, -1))


def kernel(X, edge_index, W1, b1, W2, b2):
  n, _ = X.shape
  e = edge_index.shape[1]

  src = edge_index[0].astype(jnp.int32)
  dst = edge_index[1].astype(jnp.int32)

  per_tile = -(-e // (_NW * _CHUNK))
  e_pad = per_tile * _NW * _CHUNK
  n_pad = ((n + 1 + _NS * 8 - 1) // (_NS * 8)) * (_NS * 8)
  if e_pad > e:
    src = jnp.concatenate([src, jnp.zeros((e_pad - e,), jnp.int32)])
    dst = jnp.concatenate([dst, jnp.full((e_pad - e,), n, jnp.int32)])

  degp = _make_deg_kernel(n_pad, per_tile)(dst)
  y1, dinv = _tc_prep(X, W1, degp)
  agg1 = _make_agg_kernel(n_pad, W1.shape[1], per_tile)(y1, src, dst)
  y2 = _tc_mid(agg1, y1, dinv, b1, W2)
  agg2 = _make_agg_kernel(n_pad, W2.shape[1], per_tile)(y2, src, dst)
  return _tc_final(agg2, y2, dinv, b2)
